# Initial kernel scaffold; baseline (speedup 1.0000x reference)
import jax
import jax.numpy as jnp
from jax.experimental import pallas as pl


def kernel(x, edge_index, batch, graph_features, W_emb, b_emb, Wc, bc, bn_g, bn_b, Wg1, bg1, Wg2, bg2, Wp1, bp1, Wp2, bp2, Wp3, bp3):
    raise NotImplementedError("write your pallas kernel here")


# speedup vs baseline: 10.7609x; 10.7609x over previous
import functools

import jax
import jax.numpy as jnp
from jax import lax
from jax.experimental import pallas as pl
from jax.experimental.pallas import tpu as pltpu
from jax.experimental.pallas import tpu_sc as plsc

N_NODES = 10000
N_EDGES = 320000
N_GRAPHS = 64
HID = 128
GFD = 7
BN_EPS = 1e-5

NC = 2
NS = 16
NW = NC * NS
EPW = N_EDGES // NW
CHUNK = 80
NCH = EPW // CHUNK
NNP = 10240
RPT = NNP // NS
DEGW = 8

BLK = 1000
GRID = N_NODES // BLK

_MESH = plsc.VectorSubcoreMesh(core_axis_name="c", subcore_axis_name="s")


@functools.partial(
    pl.kernel,
    out_type=jax.ShapeDtypeStruct((NC * NNP, HID), jnp.float32),
    mesh=_MESH,
    scratch_types=[
        pltpu.VMEM((CHUNK,), jnp.int32),
        pltpu.VMEM((CHUNK, HID), jnp.float32),
        pltpu.VMEM_SHARED((NNP, HID), jnp.float32),
    ],
)
def _deg_sc(dst_hbm, ones_hbm, zeros_hbm, out_hbm, idx_v, ones_v, acc_sh):
    c = lax.axis_index("c")
    s = lax.axis_index("s")
    wid = c * NS + s

    pltpu.sync_copy(zeros_hbm, ones_v)
    for t in range(RPT // CHUNK):
        pltpu.sync_copy(ones_v, acc_sh.at[pl.ds(s * RPT + t * CHUNK, CHUNK)])
    pltpu.sync_copy(ones_hbm, ones_v)
    plsc.subcore_barrier()

    def body(j, _):
        pltpu.sync_copy(dst_hbm.at[pl.ds(wid * EPW + j * CHUNK, CHUNK)], idx_v)
        pltpu.sync_copy(ones_v, acc_sh.at[idx_v], add=True)
        return 0

    lax.fori_loop(0, NCH, body, 0, unroll=False)
    plsc.subcore_barrier()
    for t in range(RPT // CHUNK):
        pltpu.sync_copy(
            acc_sh.at[pl.ds(s * RPT + t * CHUNK, CHUNK)],
            out_hbm.at[pl.ds(c * NNP + s * RPT + t * CHUNK, CHUNK)])


@functools.partial(
    pl.kernel,
    out_type=jax.ShapeDtypeStruct((NC * NNP, HID), jnp.float32),
    mesh=_MESH,
    scratch_types=[
        pltpu.VMEM((CHUNK,), jnp.int32),
        pltpu.VMEM((CHUNK,), jnp.int32),
        pltpu.VMEM((CHUNK, HID), jnp.float32),
        pltpu.VMEM_SHARED((NNP, HID), jnp.float32),
    ],
)
def _agg_sc(u_hbm, src_hbm, dst_hbm, zeros_hbm, out_hbm,
            src_v, dst_v, rows_v, acc_sh):
    c = lax.axis_index("c")
    s = lax.axis_index("s")
    wid = c * NS + s

    pltpu.sync_copy(zeros_hbm, rows_v)
    for t in range(RPT // CHUNK):
        pltpu.sync_copy(
            rows_v, acc_sh.at[pl.ds(s * RPT + t * CHUNK, CHUNK)])
    plsc.subcore_barrier()

    def body(j, _):
        base = wid * EPW + j * CHUNK
        pltpu.sync_copy(src_hbm.at[pl.ds(base, CHUNK)], src_v)
        pltpu.sync_copy(dst_hbm.at[pl.ds(base, CHUNK)], dst_v)
        pltpu.sync_copy(u_hbm.at[src_v], rows_v)
        pltpu.sync_copy(rows_v, acc_sh.at[dst_v], add=True)
        return 0

    lax.fori_loop(0, NCH, body, 0, unroll=False)

    plsc.subcore_barrier()
    for t in range(RPT // CHUNK):
        pltpu.sync_copy(
            acc_sh.at[pl.ds(s * RPT + t * CHUNK, CHUNK)],
            out_hbm.at[pl.ds(c * NNP + s * RPT + t * CHUNK, CHUNK)])


def _dinv_of(degp_ref):
    deg = degp_ref[0, :, 0:1] + degp_ref[1, :, 0:1] + 1.0
    return lax.rsqrt(deg)


def _row_spec():
    return pl.BlockSpec((BLK, HID), lambda i: (i, 0))


def _full(shape):
    nd = len(shape)
    return pl.BlockSpec(shape, lambda i: (0,) * nd)


_DEGP_SPEC = pl.BlockSpec((2, BLK, HID), lambda i: (0, i, 0))
_PARTS_SPEC = pl.BlockSpec((2, BLK, HID), lambda i: (0, i, 0))


def _tc1_body(x_ref, wemb_ref, bemb_ref, wc0_ref, degp_ref, u1_ref):
    dinv = _dinv_of(degp_ref)
    h0 = jnp.maximum(
        jnp.dot(x_ref[...], wemb_ref[...], preferred_element_type=jnp.float32)
        + bemb_ref[...], 0.0)
    u1_ref[...] = jnp.dot(
        h0, wc0_ref[...], preferred_element_type=jnp.float32) * dinv


def _tc_mid_body(parts_ref, u_ref, hprev_ref, degp_ref, sc_ref, sh_ref,
                 wnext_ref, h_ref, unext_ref, *, residual):
    dinv = _dinv_of(degp_ref)
    agg = (parts_ref[0] + parts_ref[1] + u_ref[...]) * dinv
    h = jnp.maximum(agg * sc_ref[...] + sh_ref[...], 0.0)
    if residual:
        h = h + hprev_ref[...]
    h_ref[...] = h
    unext_ref[...] = jnp.dot(
        h, wnext_ref[...], preferred_element_type=jnp.float32) * dinv


def _tc_fin_body(parts_ref, u_ref, hprev_ref, degp_ref, sc_ref, sh_ref,
                 batch_ref, gf_ref, wg1_ref, bg1_ref, wg2_ref, bg2_ref,
                 wp1a_ref, wp1b_ref, bp1_ref, wp2_ref, bp2_ref,
                 wp3_ref, bp3_ref,
                 pooled_ref, cnts_ref, out_ref):
    i = pl.program_id(0)
    dinv = _dinv_of(degp_ref)
    agg = (parts_ref[0] + parts_ref[1] + u_ref[...]) * dinv
    h = jnp.maximum(agg * sc_ref[...] + sh_ref[...], 0.0) + hprev_ref[...]

    onehot = (batch_ref[...] ==
              lax.broadcasted_iota(jnp.int32, (BLK, N_GRAPHS), 1)
              ).astype(jnp.float32)

    @pl.when(i == 0)
    def _():
        pooled_ref[...] = jnp.zeros((N_GRAPHS, HID), jnp.float32)
        cnts_ref[...] = jnp.zeros((N_GRAPHS, HID), jnp.float32)
        out_ref[...] = jnp.zeros((N_GRAPHS, 1), jnp.float32)

    dn = (((0,), (0,)), ((), ()))
    pooled_ref[...] += lax.dot_general(
        onehot, h, dn, preferred_element_type=jnp.float32)
    cnts_ref[...] += lax.dot_general(
        onehot, jnp.ones((BLK, HID), jnp.float32), dn,
        preferred_element_type=jnp.float32)

    @pl.when(i == GRID - 1)
    def _():
        emb = pooled_ref[...] / jnp.maximum(cnts_ref[...], 1.0)
        g1 = jnp.maximum(
            jnp.dot(gf_ref[...], wg1_ref[...],
                    preferred_element_type=jnp.float32) + bg1_ref[...], 0.0)
        g2 = jnp.dot(g1, wg2_ref[...],
                     preferred_element_type=jnp.float32) + bg2_ref[...]
        o = jnp.maximum(
            jnp.dot(emb, wp1a_ref[...], preferred_element_type=jnp.float32)
            + jnp.dot(g2, wp1b_ref[...], preferred_element_type=jnp.float32)
            + bp1_ref[...], 0.0)
        o = jnp.maximum(
            jnp.dot(o, wp2_ref[...], preferred_element_type=jnp.float32)
            + bp2_ref[...], 0.0)
        out_ref[...] = jnp.dot(
            o, wp3_ref[...], preferred_element_type=jnp.float32) + bp3_ref[...]


def _tc1(x, wemb, bemb, wc0, degp):
    return pl.pallas_call(
        _tc1_body,
        grid=(GRID,),
        in_specs=[_row_spec(), _full((HID, HID)), _full((1, HID)),
                  _full((HID, HID)), _DEGP_SPEC],
        out_specs=_row_spec(),
        out_shape=jax.ShapeDtypeStruct((N_NODES, HID), jnp.float32),
    )(x, wemb, bemb, wc0, degp)


def _tc_mid(parts, u, hprev, degp, scv, shv, wnext, residual):
    body = functools.partial(_tc_mid_body, residual=residual)
    return pl.pallas_call(
        body,
        grid=(GRID,),
        in_specs=[_PARTS_SPEC, _row_spec(), _row_spec(), _DEGP_SPEC,
                  _full((1, HID)), _full((1, HID)), _full((HID, HID))],
        out_specs=[_row_spec(), _row_spec()],
        out_shape=[jax.ShapeDtypeStruct((N_NODES, HID), jnp.float32),
                   jax.ShapeDtypeStruct((N_NODES, HID), jnp.float32)],
    )(parts, u, hprev, degp, scv, shv, wnext)


def _tc_fin(parts, u, hprev, degp, scv, shv, batch2, gfeat,
            wg1, bg1, wg2, bg2, wp1a, wp1b, bp1, wp2, bp2, wp3, bp3):
    h2 = HID // 2
    h4 = HID // 4
    return pl.pallas_call(
        _tc_fin_body,
        grid=(GRID,),
        in_specs=[_PARTS_SPEC, _row_spec(), _row_spec(), _DEGP_SPEC,
                  _full((1, HID)), _full((1, HID)),
                  pl.BlockSpec((BLK, 1), lambda i: (i, 0)),
                  _full((N_GRAPHS, GFD)),
                  _full((GFD, h2)), _full((1, h2)),
                  _full((h2, h4)), _full((1, h4)),
                  _full((HID, h2)), _full((h4, h2)), _full((1, h2)),
                  _full((h2, h4)), _full((1, h4)),
                  _full((h4, 1)), _full((1, 1))],
        out_specs=[_full((N_GRAPHS, HID)), _full((N_GRAPHS, HID)),
                   _full((N_GRAPHS, 1))],
        out_shape=[jax.ShapeDtypeStruct((N_GRAPHS, HID), jnp.float32),
                   jax.ShapeDtypeStruct((N_GRAPHS, HID), jnp.float32),
                   jax.ShapeDtypeStruct((N_GRAPHS, 1), jnp.float32)],
    )(parts, u, hprev, degp, scv, shv, batch2, gfeat,
      wg1, bg1, wg2, bg2, wp1a, wp1b, bp1, wp2, bp2, wp3, bp3)[2]


def kernel(x, edge_index, batch, graph_features, W_emb, b_emb, Wc, bc,
           bn_g, bn_b, Wg1, bg1, Wg2, bg2, Wp1, bp1, Wp2, bp2, Wp3, bp3):
    src1 = edge_index[0].astype(jnp.int32)
    dst1 = edge_index[1].astype(jnp.int32)
    batch2 = batch.astype(jnp.int32).reshape(N_NODES, 1)

    inv = 1.0 / jnp.sqrt(1.0 + BN_EPS)
    scv = (inv * bn_g).reshape(3, 1, HID)
    shv = (bc * inv * bn_g + bn_b).reshape(3, 1, HID)

    degp = _deg_sc(dst1, jnp.ones((CHUNK, HID), jnp.float32),
                   jnp.zeros((CHUNK, HID), jnp.float32)).reshape(2, NNP, HID)

    zrows = jnp.zeros((CHUNK, HID), jnp.float32)
    u1 = _tc1(x, W_emb, b_emb.reshape(1, HID), Wc[0], degp)
    p1 = _agg_sc(u1, src1, dst1, zrows).reshape(2, NNP, HID)
    h1, u2 = _tc_mid(p1, u1, u1, degp, scv[0], shv[0], Wc[1], residual=False)
    p2 = _agg_sc(u2, src1, dst1, zrows).reshape(2, NNP, HID)
    h2, u3 = _tc_mid(p2, u2, h1, degp, scv[1], shv[1], Wc[2], residual=True)
    p3 = _agg_sc(u3, src1, dst1, zrows).reshape(2, NNP, HID)

    return _tc_fin(p3, u3, h2, degp, scv[2], shv[2], batch2, graph_features,
                   Wg1, bg1.reshape(1, HID // 2), Wg2, bg2.reshape(1, HID // 4),
                   Wp1[:HID], Wp1[HID:], bp1.reshape(1, HID // 2),
                   Wp2, bp2.reshape(1, HID // 4), Wp3, bp3.reshape(1, 1))


# speedup vs baseline: 16.0749x; 1.4938x over previous
import functools

import jax
import jax.numpy as jnp
from jax import lax
from jax.experimental import pallas as pl
from jax.experimental.pallas import tpu as pltpu
from jax.experimental.pallas import tpu_sc as plsc

N_NODES = 10000
N_EDGES = 320000
N_GRAPHS = 64
HID = 128
GFD = 7
BN_EPS = 1e-5

NC = 2
NS = 16
NW = NC * NS
EPW = N_EDGES // NW
CHUNK = 80
NCH = EPW // CHUNK
NNP = 10240
RPT = NNP // NS
DEGW = 8

BLK = 1000
GRID = N_NODES // BLK

_MESH = plsc.VectorSubcoreMesh(core_axis_name="c", subcore_axis_name="s")


@functools.partial(
    pl.kernel,
    out_type=jax.ShapeDtypeStruct((NC * NNP, HID), jnp.float32),
    mesh=_MESH,
    scratch_types=[
        pltpu.VMEM((CHUNK,), jnp.int32),
        pltpu.VMEM((CHUNK, HID), jnp.float32),
        pltpu.VMEM_SHARED((NNP, HID), jnp.float32),
    ],
)
def _deg_sc(dst_hbm, ones_hbm, zeros_hbm, out_hbm, idx_v, ones_v, acc_sh):
    c = lax.axis_index("c")
    s = lax.axis_index("s")
    wid = c * NS + s

    pltpu.sync_copy(zeros_hbm, ones_v)
    for t in range(RPT // CHUNK):
        pltpu.sync_copy(ones_v, acc_sh.at[pl.ds(s * RPT + t * CHUNK, CHUNK)])
    pltpu.sync_copy(ones_hbm, ones_v)
    plsc.subcore_barrier()

    def body(j, _):
        pltpu.sync_copy(dst_hbm.at[pl.ds(wid * EPW + j * CHUNK, CHUNK)], idx_v)
        pltpu.sync_copy(ones_v, acc_sh.at[idx_v], add=True)
        return 0

    lax.fori_loop(0, NCH, body, 0, unroll=False)
    plsc.subcore_barrier()
    for t in range(RPT // CHUNK):
        pltpu.sync_copy(
            acc_sh.at[pl.ds(s * RPT + t * CHUNK, CHUNK)],
            out_hbm.at[pl.ds(c * NNP + s * RPT + t * CHUNK, CHUNK)])


@functools.partial(
    pl.kernel,
    out_type=jax.ShapeDtypeStruct((NC * NNP, HID), jnp.float32),
    mesh=_MESH,
    scratch_types=[
        pltpu.VMEM((CHUNK,), jnp.int32),
        pltpu.VMEM((CHUNK,), jnp.int32),
        pltpu.VMEM((CHUNK,), jnp.int32),
        pltpu.VMEM((CHUNK,), jnp.int32),
        pltpu.VMEM((CHUNK, HID), jnp.float32),
        pltpu.VMEM((CHUNK, HID), jnp.float32),
        pltpu.VMEM_SHARED((NNP, HID), jnp.float32),
        pltpu.SemaphoreType.DMA,
        pltpu.SemaphoreType.DMA,
    ],
)
def _agg_sc(u_hbm, src_hbm, dst_hbm, zeros_hbm, out_hbm,
            s0, s1, d0, d1, r0, r1, acc_sh, g0, g1):
    c = lax.axis_index("c")
    s = lax.axis_index("s")
    wid = c * NS + s

    pltpu.sync_copy(zeros_hbm, r0)
    for t in range(RPT // CHUNK):
        pltpu.sync_copy(
            r0, acc_sh.at[pl.ds(s * RPT + t * CHUNK, CHUNK)])
    plsc.subcore_barrier()

    sv = (s0, s1)
    dv = (d0, d1)
    rv = (r0, r1)
    gs = (g0, g1)

    def fetch(j, b):
        base = wid * EPW + j * CHUNK
        pltpu.sync_copy(src_hbm.at[pl.ds(base, CHUNK)], sv[b])
        pltpu.sync_copy(dst_hbm.at[pl.ds(base, CHUNK)], dv[b])
        pltpu.async_copy(u_hbm.at[sv[b]], rv[b], gs[b])

    fetch(0, 0)

    def body(jj, _):
        for b in range(2):
            j = 2 * jj + b

            @pl.when(j < NCH)
            def _():
                @pl.when(j + 1 < NCH)
                def _():
                    fetch(j + 1, 1 - b)
                pltpu.make_async_copy(u_hbm.at[sv[b]], rv[b], gs[b]).wait()
                pltpu.sync_copy(rv[b], acc_sh.at[dv[b]], add=True)
        return 0

    lax.fori_loop(0, (NCH + 1) // 2, body, 0, unroll=False)

    plsc.subcore_barrier()
    for t in range(RPT // CHUNK):
        pltpu.sync_copy(
            acc_sh.at[pl.ds(s * RPT + t * CHUNK, CHUNK)],
            out_hbm.at[pl.ds(c * NNP + s * RPT + t * CHUNK, CHUNK)])


def _dinv_of(degp_ref):
    deg = degp_ref[0, :, 0:1] + degp_ref[1, :, 0:1] + 1.0
    return lax.rsqrt(deg)


def _row_spec():
    return pl.BlockSpec((BLK, HID), lambda i: (i, 0))


def _full(shape):
    nd = len(shape)
    return pl.BlockSpec(shape, lambda i: (0,) * nd)


_DEGP_SPEC = pl.BlockSpec((2, BLK, HID), lambda i: (0, i, 0))
_PARTS_SPEC = pl.BlockSpec((2, BLK, HID), lambda i: (0, i, 0))


def _tc1_body(x_ref, wemb_ref, bemb_ref, wc0_ref, degp_ref, u1_ref):
    dinv = _dinv_of(degp_ref)
    h0 = jnp.maximum(
        jnp.dot(x_ref[...], wemb_ref[...], preferred_element_type=jnp.float32)
        + bemb_ref[...], 0.0)
    u1_ref[...] = jnp.dot(
        h0, wc0_ref[...], preferred_element_type=jnp.float32) * dinv


def _tc_mid_body(parts_ref, u_ref, hprev_ref, degp_ref, sc_ref, sh_ref,
                 wnext_ref, h_ref, unext_ref, *, residual):
    dinv = _dinv_of(degp_ref)
    agg = (parts_ref[0] + parts_ref[1] + u_ref[...]) * dinv
    h = jnp.maximum(agg * sc_ref[...] + sh_ref[...], 0.0)
    if residual:
        h = h + hprev_ref[...]
    h_ref[...] = h
    unext_ref[...] = jnp.dot(
        h, wnext_ref[...], preferred_element_type=jnp.float32) * dinv


def _tc_fin_body(parts_ref, u_ref, hprev_ref, degp_ref, sc_ref, sh_ref,
                 batch_ref, gf_ref, wg1_ref, bg1_ref, wg2_ref, bg2_ref,
                 wp1a_ref, wp1b_ref, bp1_ref, wp2_ref, bp2_ref,
                 wp3_ref, bp3_ref,
                 pooled_ref, cnts_ref, out_ref):
    i = pl.program_id(0)
    dinv = _dinv_of(degp_ref)
    agg = (parts_ref[0] + parts_ref[1] + u_ref[...]) * dinv
    h = jnp.maximum(agg * sc_ref[...] + sh_ref[...], 0.0) + hprev_ref[...]

    onehot = (batch_ref[...] ==
              lax.broadcasted_iota(jnp.int32, (BLK, N_GRAPHS), 1)
              ).astype(jnp.float32)

    @pl.when(i == 0)
    def _():
        pooled_ref[...] = jnp.zeros((N_GRAPHS, HID), jnp.float32)
        cnts_ref[...] = jnp.zeros((N_GRAPHS, HID), jnp.float32)
        out_ref[...] = jnp.zeros((N_GRAPHS, 1), jnp.float32)

    dn = (((0,), (0,)), ((), ()))
    pooled_ref[...] += lax.dot_general(
        onehot, h, dn, preferred_element_type=jnp.float32)
    cnts_ref[...] += lax.dot_general(
        onehot, jnp.ones((BLK, HID), jnp.float32), dn,
        preferred_element_type=jnp.float32)

    @pl.when(i == GRID - 1)
    def _():
        emb = pooled_ref[...] / jnp.maximum(cnts_ref[...], 1.0)
        g1 = jnp.maximum(
            jnp.dot(gf_ref[...], wg1_ref[...],
                    preferred_element_type=jnp.float32) + bg1_ref[...], 0.0)
        g2 = jnp.dot(g1, wg2_ref[...],
                     preferred_element_type=jnp.float32) + bg2_ref[...]
        o = jnp.maximum(
            jnp.dot(emb, wp1a_ref[...], preferred_element_type=jnp.float32)
            + jnp.dot(g2, wp1b_ref[...], preferred_element_type=jnp.float32)
            + bp1_ref[...], 0.0)
        o = jnp.maximum(
            jnp.dot(o, wp2_ref[...], preferred_element_type=jnp.float32)
            + bp2_ref[...], 0.0)
        out_ref[...] = jnp.dot(
            o, wp3_ref[...], preferred_element_type=jnp.float32) + bp3_ref[...]


def _tc1(x, wemb, bemb, wc0, degp):
    return pl.pallas_call(
        _tc1_body,
        grid=(GRID,),
        in_specs=[_row_spec(), _full((HID, HID)), _full((1, HID)),
                  _full((HID, HID)), _DEGP_SPEC],
        out_specs=_row_spec(),
        out_shape=jax.ShapeDtypeStruct((N_NODES, HID), jnp.float32),
    )(x, wemb, bemb, wc0, degp)


def _tc_mid(parts, u, hprev, degp, scv, shv, wnext, residual):
    body = functools.partial(_tc_mid_body, residual=residual)
    return pl.pallas_call(
        body,
        grid=(GRID,),
        in_specs=[_PARTS_SPEC, _row_spec(), _row_spec(), _DEGP_SPEC,
                  _full((1, HID)), _full((1, HID)), _full((HID, HID))],
        out_specs=[_row_spec(), _row_spec()],
        out_shape=[jax.ShapeDtypeStruct((N_NODES, HID), jnp.float32),
                   jax.ShapeDtypeStruct((N_NODES, HID), jnp.float32)],
    )(parts, u, hprev, degp, scv, shv, wnext)


def _tc_fin(parts, u, hprev, degp, scv, shv, batch2, gfeat,
            wg1, bg1, wg2, bg2, wp1a, wp1b, bp1, wp2, bp2, wp3, bp3):
    h2 = HID // 2
    h4 = HID // 4
    return pl.pallas_call(
        _tc_fin_body,
        grid=(GRID,),
        in_specs=[_PARTS_SPEC, _row_spec(), _row_spec(), _DEGP_SPEC,
                  _full((1, HID)), _full((1, HID)),
                  pl.BlockSpec((BLK, 1), lambda i: (i, 0)),
                  _full((N_GRAPHS, GFD)),
                  _full((GFD, h2)), _full((1, h2)),
                  _full((h2, h4)), _full((1, h4)),
                  _full((HID, h2)), _full((h4, h2)), _full((1, h2)),
                  _full((h2, h4)), _full((1, h4)),
                  _full((h4, 1)), _full((1, 1))],
        out_specs=[_full((N_GRAPHS, HID)), _full((N_GRAPHS, HID)),
                   _full((N_GRAPHS, 1))],
        out_shape=[jax.ShapeDtypeStruct((N_GRAPHS, HID), jnp.float32),
                   jax.ShapeDtypeStruct((N_GRAPHS, HID), jnp.float32),
                   jax.ShapeDtypeStruct((N_GRAPHS, 1), jnp.float32)],
    )(parts, u, hprev, degp, scv, shv, batch2, gfeat,
      wg1, bg1, wg2, bg2, wp1a, wp1b, bp1, wp2, bp2, wp3, bp3)[2]


def kernel(x, edge_index, batch, graph_features, W_emb, b_emb, Wc, bc,
           bn_g, bn_b, Wg1, bg1, Wg2, bg2, Wp1, bp1, Wp2, bp2, Wp3, bp3):
    src1 = edge_index[0].astype(jnp.int32)
    dst1 = edge_index[1].astype(jnp.int32)
    batch2 = batch.astype(jnp.int32).reshape(N_NODES, 1)

    inv = 1.0 / jnp.sqrt(1.0 + BN_EPS)
    scv = (inv * bn_g).reshape(3, 1, HID)
    shv = (bc * inv * bn_g + bn_b).reshape(3, 1, HID)

    degp = _deg_sc(dst1, jnp.ones((CHUNK, HID), jnp.float32),
                   jnp.zeros((CHUNK, HID), jnp.float32)).reshape(2, NNP, HID)

    zrows = jnp.zeros((CHUNK, HID), jnp.float32)
    u1 = _tc1(x, W_emb, b_emb.reshape(1, HID), Wc[0], degp)
    p1 = _agg_sc(u1, src1, dst1, zrows).reshape(2, NNP, HID)
    h1, u2 = _tc_mid(p1, u1, u1, degp, scv[0], shv[0], Wc[1], residual=False)
    p2 = _agg_sc(u2, src1, dst1, zrows).reshape(2, NNP, HID)
    h2, u3 = _tc_mid(p2, u2, h1, degp, scv[1], shv[1], Wc[2], residual=True)
    p3 = _agg_sc(u3, src1, dst1, zrows).reshape(2, NNP, HID)

    return _tc_fin(p3, u3, h2, degp, scv[2], shv[2], batch2, graph_features,
                   Wg1, bg1.reshape(1, HID // 2), Wg2, bg2.reshape(1, HID // 4),
                   Wp1[:HID], Wp1[HID:], bp1.reshape(1, HID // 2),
                   Wp2, bp2.reshape(1, HID // 4), Wp3, bp3.reshape(1, 1))


# speedup vs baseline: 17.1735x; 1.0683x over previous
import functools

import jax
import jax.numpy as jnp
from jax import lax
from jax.experimental import pallas as pl
from jax.experimental.pallas import tpu as pltpu
from jax.experimental.pallas import tpu_sc as plsc

N_NODES = 10000
N_EDGES = 320000
N_GRAPHS = 64
HID = 128
GFD = 7
BN_EPS = 1e-5

NC = 2
NS = 16
NW = NC * NS
EPW = N_EDGES // NW
CHUNK = 80
NCH = EPW // CHUNK
NNP = 10240
RPT = NNP // NS
DEGW = 8

BLK = 1000
GRID = N_NODES // BLK

_MESH = plsc.VectorSubcoreMesh(core_axis_name="c", subcore_axis_name="s")


@functools.partial(
    pl.kernel,
    out_type=jax.ShapeDtypeStruct((NC * NNP, HID), jnp.float32),
    mesh=_MESH,
    scratch_types=[
        pltpu.VMEM((CHUNK,), jnp.int32),
        pltpu.VMEM((CHUNK,), jnp.int32),
        pltpu.VMEM((CHUNK, HID), jnp.float32),
        pltpu.VMEM_SHARED((NNP, HID), jnp.float32),
        pltpu.SemaphoreType.DMA,
        pltpu.SemaphoreType.DMA,
    ],
)
def _deg_sc(dst_hbm, ones_hbm, zeros_hbm, out_hbm,
            i0, i1, ones_v, acc_sh, m0, m1):
    c = lax.axis_index("c")
    s = lax.axis_index("s")
    wid = c * NS + s

    pltpu.sync_copy(zeros_hbm, ones_v)
    for t in range(RPT // CHUNK):
        pltpu.sync_copy(ones_v, acc_sh.at[pl.ds(s * RPT + t * CHUNK, CHUNK)])
    pltpu.sync_copy(ones_hbm, ones_v)
    plsc.subcore_barrier()

    iv = (i0, i1)
    ms = (m0, m1)

    def fetch(j, b):
        pltpu.async_copy(dst_hbm.at[pl.ds(wid * EPW + j * CHUNK, CHUNK)],
                         iv[b], ms[b])

    fetch(0, 0)

    def body(jj, _):
        for b in range(2):
            j = 2 * jj + b

            @pl.when(j < NCH)
            def _():
                @pl.when(j + 1 < NCH)
                def _():
                    fetch(j + 1, 1 - b)
                pltpu.make_async_copy(
                    dst_hbm.at[pl.ds(wid * EPW + j * CHUNK, CHUNK)],
                    iv[b], ms[b]).wait()
                pltpu.sync_copy(ones_v, acc_sh.at[iv[b]], add=True)
        return 0

    lax.fori_loop(0, (NCH + 1) // 2, body, 0, unroll=False)
    plsc.subcore_barrier()
    for t in range(RPT // CHUNK):
        pltpu.sync_copy(
            acc_sh.at[pl.ds(s * RPT + t * CHUNK, CHUNK)],
            out_hbm.at[pl.ds(c * NNP + s * RPT + t * CHUNK, CHUNK)])


@functools.partial(
    pl.kernel,
    out_type=jax.ShapeDtypeStruct((NC * NNP, HID), jnp.float32),
    mesh=_MESH,
    scratch_types=[
        pltpu.VMEM((CHUNK,), jnp.int32),
        pltpu.VMEM((CHUNK,), jnp.int32),
        pltpu.VMEM((CHUNK,), jnp.int32),
        pltpu.VMEM((CHUNK,), jnp.int32),
        pltpu.VMEM((CHUNK, HID), jnp.float32),
        pltpu.VMEM((CHUNK, HID), jnp.float32),
        pltpu.VMEM_SHARED((NNP, HID), jnp.float32),
        pltpu.SemaphoreType.DMA,
        pltpu.SemaphoreType.DMA,
    ],
)
def _agg_sc(u_hbm, src_hbm, dst_hbm, zeros_hbm, out_hbm,
            s0, s1, d0, d1, r0, r1, acc_sh, g0, g1):
    c = lax.axis_index("c")
    s = lax.axis_index("s")
    wid = c * NS + s

    pltpu.sync_copy(zeros_hbm, r0)
    for t in range(RPT // CHUNK):
        pltpu.sync_copy(
            r0, acc_sh.at[pl.ds(s * RPT + t * CHUNK, CHUNK)])
    plsc.subcore_barrier()

    sv = (s0, s1)
    dv = (d0, d1)
    rv = (r0, r1)
    gs = (g0, g1)

    def fetch(j, b):
        base = wid * EPW + j * CHUNK
        pltpu.sync_copy(src_hbm.at[pl.ds(base, CHUNK)], sv[b])
        pltpu.sync_copy(dst_hbm.at[pl.ds(base, CHUNK)], dv[b])
        pltpu.async_copy(u_hbm.at[sv[b]], rv[b], gs[b])

    fetch(0, 0)

    def body(jj, _):
        for b in range(2):
            j = 2 * jj + b

            @pl.when(j < NCH)
            def _():
                @pl.when(j + 1 < NCH)
                def _():
                    fetch(j + 1, 1 - b)
                pltpu.make_async_copy(u_hbm.at[sv[b]], rv[b], gs[b]).wait()
                pltpu.sync_copy(rv[b], acc_sh.at[dv[b]], add=True)
        return 0

    lax.fori_loop(0, (NCH + 1) // 2, body, 0, unroll=False)

    plsc.subcore_barrier()
    for t in range(RPT // CHUNK):
        pltpu.sync_copy(
            acc_sh.at[pl.ds(s * RPT + t * CHUNK, CHUNK)],
            out_hbm.at[pl.ds(c * NNP + s * RPT + t * CHUNK, CHUNK)])


def _dinv_of(degp_ref):
    deg = degp_ref[0, :, 0:1] + degp_ref[1, :, 0:1] + 1.0
    return lax.rsqrt(deg)


def _row_spec():
    return pl.BlockSpec((BLK, HID), lambda i: (i, 0))


def _full(shape):
    nd = len(shape)
    return pl.BlockSpec(shape, lambda i: (0,) * nd)


_DEGP_SPEC = pl.BlockSpec((2, BLK, HID), lambda i: (0, i, 0))
_PARTS_SPEC = pl.BlockSpec((2, BLK, HID), lambda i: (0, i, 0))


def _tc_emb_body(x_ref, wemb_ref, bemb_ref, h0_ref):
    h0_ref[...] = jnp.maximum(
        jnp.dot(x_ref[...], wemb_ref[...], preferred_element_type=jnp.float32)
        + bemb_ref[...], 0.0)


def _tc1_body(h0_ref, wc0_ref, degp_ref, u1_ref):
    dinv = _dinv_of(degp_ref)
    u1_ref[...] = jnp.dot(
        h0_ref[...], wc0_ref[...], preferred_element_type=jnp.float32) * dinv


def _tc_mid_body(parts_ref, u_ref, hprev_ref, degp_ref, sc_ref, sh_ref,
                 wnext_ref, h_ref, unext_ref, *, residual):
    dinv = _dinv_of(degp_ref)
    agg = (parts_ref[0] + parts_ref[1] + u_ref[...]) * dinv
    h = jnp.maximum(agg * sc_ref[...] + sh_ref[...], 0.0)
    if residual:
        h = h + hprev_ref[...]
    h_ref[...] = h
    unext_ref[...] = jnp.dot(
        h, wnext_ref[...], preferred_element_type=jnp.float32) * dinv


def _tc_fin_body(parts_ref, u_ref, hprev_ref, degp_ref, sc_ref, sh_ref,
                 batch_ref, gf_ref, wg1_ref, bg1_ref, wg2_ref, bg2_ref,
                 wp1a_ref, wp1b_ref, bp1_ref, wp2_ref, bp2_ref,
                 wp3_ref, bp3_ref,
                 pooled_ref, cnts_ref, out_ref):
    i = pl.program_id(0)
    dinv = _dinv_of(degp_ref)
    agg = (parts_ref[0] + parts_ref[1] + u_ref[...]) * dinv
    h = jnp.maximum(agg * sc_ref[...] + sh_ref[...], 0.0) + hprev_ref[...]

    onehot = (batch_ref[...] ==
              lax.broadcasted_iota(jnp.int32, (BLK, N_GRAPHS), 1)
              ).astype(jnp.float32)

    @pl.when(i == 0)
    def _():
        pooled_ref[...] = jnp.zeros((N_GRAPHS, HID), jnp.float32)
        cnts_ref[...] = jnp.zeros((N_GRAPHS, HID), jnp.float32)
        out_ref[...] = jnp.zeros((N_GRAPHS, 1), jnp.float32)

    dn = (((0,), (0,)), ((), ()))
    pooled_ref[...] += lax.dot_general(
        onehot, h, dn, preferred_element_type=jnp.float32)
    cnts_ref[...] += lax.dot_general(
        onehot, jnp.ones((BLK, HID), jnp.float32), dn,
        preferred_element_type=jnp.float32)

    @pl.when(i == GRID - 1)
    def _():
        emb = pooled_ref[...] / jnp.maximum(cnts_ref[...], 1.0)
        g1 = jnp.maximum(
            jnp.dot(gf_ref[...], wg1_ref[...],
                    preferred_element_type=jnp.float32) + bg1_ref[...], 0.0)
        g2 = jnp.dot(g1, wg2_ref[...],
                     preferred_element_type=jnp.float32) + bg2_ref[...]
        o = jnp.maximum(
            jnp.dot(emb, wp1a_ref[...], preferred_element_type=jnp.float32)
            + jnp.dot(g2, wp1b_ref[...], preferred_element_type=jnp.float32)
            + bp1_ref[...], 0.0)
        o = jnp.maximum(
            jnp.dot(o, wp2_ref[...], preferred_element_type=jnp.float32)
            + bp2_ref[...], 0.0)
        out_ref[...] = jnp.dot(
            o, wp3_ref[...], preferred_element_type=jnp.float32) + bp3_ref[...]


def _tc_emb(x, wemb, bemb):
    return pl.pallas_call(
        _tc_emb_body,
        grid=(GRID,),
        in_specs=[_row_spec(), _full((HID, HID)), _full((1, HID))],
        out_specs=_row_spec(),
        out_shape=jax.ShapeDtypeStruct((N_NODES, HID), jnp.float32),
    )(x, wemb, bemb)


def _tc1(h0, wc0, degp):
    return pl.pallas_call(
        _tc1_body,
        grid=(GRID,),
        in_specs=[_row_spec(), _full((HID, HID)), _DEGP_SPEC],
        out_specs=_row_spec(),
        out_shape=jax.ShapeDtypeStruct((N_NODES, HID), jnp.float32),
    )(h0, wc0, degp)


def _tc_mid(parts, u, hprev, degp, scv, shv, wnext, residual):
    body = functools.partial(_tc_mid_body, residual=residual)
    return pl.pallas_call(
        body,
        grid=(GRID,),
        in_specs=[_PARTS_SPEC, _row_spec(), _row_spec(), _DEGP_SPEC,
                  _full((1, HID)), _full((1, HID)), _full((HID, HID))],
        out_specs=[_row_spec(), _row_spec()],
        out_shape=[jax.ShapeDtypeStruct((N_NODES, HID), jnp.float32),
                   jax.ShapeDtypeStruct((N_NODES, HID), jnp.float32)],
    )(parts, u, hprev, degp, scv, shv, wnext)


def _tc_fin(parts, u, hprev, degp, scv, shv, batch2, gfeat,
            wg1, bg1, wg2, bg2, wp1a, wp1b, bp1, wp2, bp2, wp3, bp3):
    h2 = HID // 2
    h4 = HID // 4
    return pl.pallas_call(
        _tc_fin_body,
        grid=(GRID,),
        in_specs=[_PARTS_SPEC, _row_spec(), _row_spec(), _DEGP_SPEC,
                  _full((1, HID)), _full((1, HID)),
                  pl.BlockSpec((BLK, 1), lambda i: (i, 0)),
                  _full((N_GRAPHS, GFD)),
                  _full((GFD, h2)), _full((1, h2)),
                  _full((h2, h4)), _full((1, h4)),
                  _full((HID, h2)), _full((h4, h2)), _full((1, h2)),
                  _full((h2, h4)), _full((1, h4)),
                  _full((h4, 1)), _full((1, 1))],
        out_specs=[_full((N_GRAPHS, HID)), _full((N_GRAPHS, HID)),
                   _full((N_GRAPHS, 1))],
        out_shape=[jax.ShapeDtypeStruct((N_GRAPHS, HID), jnp.float32),
                   jax.ShapeDtypeStruct((N_GRAPHS, HID), jnp.float32),
                   jax.ShapeDtypeStruct((N_GRAPHS, 1), jnp.float32)],
    )(parts, u, hprev, degp, scv, shv, batch2, gfeat,
      wg1, bg1, wg2, bg2, wp1a, wp1b, bp1, wp2, bp2, wp3, bp3)[2]


def kernel(x, edge_index, batch, graph_features, W_emb, b_emb, Wc, bc,
           bn_g, bn_b, Wg1, bg1, Wg2, bg2, Wp1, bp1, Wp2, bp2, Wp3, bp3):
    src1 = edge_index[0].astype(jnp.int32)
    dst1 = edge_index[1].astype(jnp.int32)
    batch2 = batch.astype(jnp.int32).reshape(N_NODES, 1)

    inv = 1.0 / jnp.sqrt(1.0 + BN_EPS)
    scv = (inv * bn_g).reshape(3, 1, HID)
    shv = (bc * inv * bn_g + bn_b).reshape(3, 1, HID)

    degp = _deg_sc(dst1, jnp.ones((CHUNK, HID), jnp.float32),
                   jnp.zeros((CHUNK, HID), jnp.float32)).reshape(2, NNP, HID)

    zrows = jnp.zeros((CHUNK, HID), jnp.float32)
    h0 = _tc_emb(x, W_emb, b_emb.reshape(1, HID))
    u1 = _tc1(h0, Wc[0], degp)
    p1 = _agg_sc(u1, src1, dst1, zrows).reshape(2, NNP, HID)
    h1, u2 = _tc_mid(p1, u1, u1, degp, scv[0], shv[0], Wc[1], residual=False)
    p2 = _agg_sc(u2, src1, dst1, zrows).reshape(2, NNP, HID)
    h2, u3 = _tc_mid(p2, u2, h1, degp, scv[1], shv[1], Wc[2], residual=True)
    p3 = _agg_sc(u3, src1, dst1, zrows).reshape(2, NNP, HID)

    return _tc_fin(p3, u3, h2, degp, scv[2], shv[2], batch2, graph_features,
                   Wg1, bg1.reshape(1, HID // 2), Wg2, bg2.reshape(1, HID // 4),
                   Wp1[:HID], Wp1[HID:], bp1.reshape(1, HID // 2),
                   Wp2, bp2.reshape(1, HID // 4), Wp3, bp3.reshape(1, 1))


# speedup vs baseline: 20.3501x; 1.1850x over previous
import functools

import jax
import jax.numpy as jnp
from jax import lax
from jax.experimental import pallas as pl
from jax.experimental.pallas import tpu as pltpu
from jax.experimental.pallas import tpu_sc as plsc

N_NODES = 10000
N_EDGES = 320000
N_GRAPHS = 64
HID = 128
GFD = 7
BN_EPS = 1e-5

NC = 2
NS = 16
NW = NC * NS
EPW = N_EDGES // NW
CHUNK = 80
NCH = EPW // CHUNK
NNP = 10240
RPT = NNP // NS
DEGW = 8

BLK = 1000
GRID = N_NODES // BLK

_MESH = plsc.VectorSubcoreMesh(core_axis_name="c", subcore_axis_name="s")


@functools.partial(
    pl.kernel,
    out_type=jax.ShapeDtypeStruct((NC * NNP, HID), jnp.float32),
    mesh=_MESH,
    scratch_types=[
        pltpu.VMEM((CHUNK,), jnp.int32),
        pltpu.VMEM((CHUNK,), jnp.int32),
        pltpu.VMEM((CHUNK, HID), jnp.float32),
        pltpu.VMEM_SHARED((NNP, HID), jnp.float32),
        pltpu.SemaphoreType.DMA,
        pltpu.SemaphoreType.DMA,
    ],
)
def _deg_sc(dst_hbm, ones_hbm, zeros_hbm, out_hbm,
            i0, i1, ones_v, acc_sh, m0, m1):
    c = lax.axis_index("c")
    s = lax.axis_index("s")
    wid = c * NS + s

    pltpu.sync_copy(zeros_hbm, ones_v)
    for t in range(RPT // CHUNK):
        pltpu.sync_copy(ones_v, acc_sh.at[pl.ds(s * RPT + t * CHUNK, CHUNK)])
    pltpu.sync_copy(ones_hbm, ones_v)
    plsc.subcore_barrier()

    iv = (i0, i1)
    ms = (m0, m1)

    def fetch(j, b):
        pltpu.async_copy(dst_hbm.at[pl.ds(wid * EPW + j * CHUNK, CHUNK)],
                         iv[b], ms[b])

    fetch(0, 0)

    def body(jj, _):
        for b in range(2):
            j = 2 * jj + b

            @pl.when(j < NCH)
            def _():
                @pl.when(j + 1 < NCH)
                def _():
                    fetch(j + 1, 1 - b)
                pltpu.make_async_copy(
                    dst_hbm.at[pl.ds(wid * EPW + j * CHUNK, CHUNK)],
                    iv[b], ms[b]).wait()
                pltpu.sync_copy(ones_v, acc_sh.at[iv[b]], add=True)
        return 0

    lax.fori_loop(0, (NCH + 1) // 2, body, 0, unroll=False)
    plsc.subcore_barrier()
    for t in range(RPT // CHUNK):
        pltpu.sync_copy(
            acc_sh.at[pl.ds(s * RPT + t * CHUNK, CHUNK)],
            out_hbm.at[pl.ds(c * NNP + s * RPT + t * CHUNK, CHUNK)])


@functools.partial(
    pl.kernel,
    out_type=jax.ShapeDtypeStruct((NC * NNP, HID), jnp.float32),
    mesh=_MESH,
    scratch_types=[
        pltpu.VMEM((CHUNK,), jnp.int32),
        pltpu.VMEM((CHUNK,), jnp.int32),
        pltpu.VMEM((CHUNK,), jnp.int32),
        pltpu.VMEM((CHUNK,), jnp.int32),
        pltpu.VMEM((CHUNK, HID), jnp.float32),
        pltpu.VMEM((CHUNK, HID), jnp.float32),
        pltpu.VMEM_SHARED((NNP, HID), jnp.float32),
        pltpu.SemaphoreType.DMA,
        pltpu.SemaphoreType.DMA,
        pltpu.SemaphoreType.DMA,
        pltpu.SemaphoreType.DMA,
    ],
)
def _agg_sc(u_hbm, src_hbm, dst_hbm, zeros_hbm, out_hbm,
            s0, s1, d0, d1, r0, r1, acc_sh, g0, g1, m0, m1):
    c = lax.axis_index("c")
    s = lax.axis_index("s")
    wid = c * NS + s

    pltpu.sync_copy(zeros_hbm, r0)
    for t in range(RPT // CHUNK):
        pltpu.sync_copy(
            r0, acc_sh.at[pl.ds(s * RPT + t * CHUNK, CHUNK)])
    plsc.subcore_barrier()

    sv = (s0, s1)
    dv = (d0, d1)
    rv = (r0, r1)
    gs = (g0, g1)
    ms = (m0, m1)

    def fetch_idx(j, b):
        base = wid * EPW + j * CHUNK
        pltpu.async_copy(src_hbm.at[pl.ds(base, CHUNK)], sv[b], ms[b])
        pltpu.async_copy(dst_hbm.at[pl.ds(base, CHUNK)], dv[b], ms[b])

    def wait_idx(j, b):
        base = wid * EPW + j * CHUNK
        pltpu.make_async_copy(src_hbm.at[pl.ds(base, CHUNK)], sv[b], ms[b]).wait()
        pltpu.make_async_copy(dst_hbm.at[pl.ds(base, CHUNK)], dv[b], ms[b]).wait()

    fetch_idx(0, 0)
    fetch_idx(1, 1)
    wait_idx(0, 0)
    pltpu.async_copy(u_hbm.at[sv[0]], rv[0], gs[0])

    def body(jj, _):
        for b in range(2):
            j = 2 * jj + b

            @pl.when(j < NCH)
            def _():
                @pl.when(j + 1 < NCH)
                def _():
                    wait_idx(j + 1, 1 - b)
                    pltpu.async_copy(u_hbm.at[sv[1 - b]], rv[1 - b], gs[1 - b])
                pltpu.make_async_copy(u_hbm.at[sv[b]], rv[b], gs[b]).wait()
                pltpu.sync_copy(rv[b], acc_sh.at[dv[b]], add=True)

                @pl.when(j + 2 < NCH)
                def _():
                    fetch_idx(j + 2, b)
        return 0

    lax.fori_loop(0, (NCH + 1) // 2, body, 0, unroll=False)

    plsc.subcore_barrier()
    for t in range(RPT // CHUNK):
        pltpu.sync_copy(
            acc_sh.at[pl.ds(s * RPT + t * CHUNK, CHUNK)],
            out_hbm.at[pl.ds(c * NNP + s * RPT + t * CHUNK, CHUNK)])


def _dinv_of(degp_ref):
    deg = degp_ref[0, :, 0:1] + degp_ref[1, :, 0:1] + 1.0
    return lax.rsqrt(deg)


def _row_spec():
    return pl.BlockSpec((BLK, HID), lambda i: (i, 0))


def _full(shape):
    nd = len(shape)
    return pl.BlockSpec(shape, lambda i: (0,) * nd)


_DEGP_SPEC = pl.BlockSpec((2, BLK, HID), lambda i: (0, i, 0))
_PARTS_SPEC = pl.BlockSpec((2, BLK, HID), lambda i: (0, i, 0))


def _tc_emb_body(x_ref, wemb_ref, bemb_ref, h0_ref):
    h0_ref[...] = jnp.maximum(
        jnp.dot(x_ref[...], wemb_ref[...], preferred_element_type=jnp.float32)
        + bemb_ref[...], 0.0)


def _tc1_body(h0_ref, wc0_ref, degp_ref, u1_ref):
    dinv = _dinv_of(degp_ref)
    u1_ref[...] = jnp.dot(
        h0_ref[...], wc0_ref[...], preferred_element_type=jnp.float32) * dinv


def _tc_mid_body(parts_ref, u_ref, hprev_ref, degp_ref, sc_ref, sh_ref,
                 wnext_ref, h_ref, unext_ref, *, residual):
    dinv = _dinv_of(degp_ref)
    agg = (parts_ref[0] + parts_ref[1] + u_ref[...]) * dinv
    h = jnp.maximum(agg * sc_ref[...] + sh_ref[...], 0.0)
    if residual:
        h = h + hprev_ref[...]
    h_ref[...] = h
    unext_ref[...] = jnp.dot(
        h, wnext_ref[...], preferred_element_type=jnp.float32) * dinv


def _tc_fin_body(parts_ref, u_ref, hprev_ref, degp_ref, sc_ref, sh_ref,
                 batch_ref, gf_ref, wg1_ref, bg1_ref, wg2_ref, bg2_ref,
                 wp1a_ref, wp1b_ref, bp1_ref, wp2_ref, bp2_ref,
                 wp3_ref, bp3_ref,
                 pooled_ref, cnts_ref, out_ref):
    i = pl.program_id(0)
    dinv = _dinv_of(degp_ref)
    agg = (parts_ref[0] + parts_ref[1] + u_ref[...]) * dinv
    h = jnp.maximum(agg * sc_ref[...] + sh_ref[...], 0.0) + hprev_ref[...]

    onehot = (batch_ref[...] ==
              lax.broadcasted_iota(jnp.int32, (BLK, N_GRAPHS), 1)
              ).astype(jnp.float32)

    @pl.when(i == 0)
    def _():
        pooled_ref[...] = jnp.zeros((N_GRAPHS, HID), jnp.float32)
        cnts_ref[...] = jnp.zeros((N_GRAPHS, HID), jnp.float32)
        out_ref[...] = jnp.zeros((N_GRAPHS, 1), jnp.float32)

    dn = (((0,), (0,)), ((), ()))
    pooled_ref[...] += lax.dot_general(
        onehot, h, dn, preferred_element_type=jnp.float32)
    cnts_ref[...] += lax.dot_general(
        onehot, jnp.ones((BLK, HID), jnp.float32), dn,
        preferred_element_type=jnp.float32)

    @pl.when(i == GRID - 1)
    def _():
        emb = pooled_ref[...] / jnp.maximum(cnts_ref[...], 1.0)
        g1 = jnp.maximum(
            jnp.dot(gf_ref[...], wg1_ref[...],
                    preferred_element_type=jnp.float32) + bg1_ref[...], 0.0)
        g2 = jnp.dot(g1, wg2_ref[...],
                     preferred_element_type=jnp.float32) + bg2_ref[...]
        o = jnp.maximum(
            jnp.dot(emb, wp1a_ref[...], preferred_element_type=jnp.float32)
            + jnp.dot(g2, wp1b_ref[...], preferred_element_type=jnp.float32)
            + bp1_ref[...], 0.0)
        o = jnp.maximum(
            jnp.dot(o, wp2_ref[...], preferred_element_type=jnp.float32)
            + bp2_ref[...], 0.0)
        out_ref[...] = jnp.dot(
            o, wp3_ref[...], preferred_element_type=jnp.float32) + bp3_ref[...]


def _tc_emb(x, wemb, bemb):
    return pl.pallas_call(
        _tc_emb_body,
        grid=(GRID,),
        in_specs=[_row_spec(), _full((HID, HID)), _full((1, HID))],
        out_specs=_row_spec(),
        out_shape=jax.ShapeDtypeStruct((N_NODES, HID), jnp.float32),
    )(x, wemb, bemb)


def _tc1(h0, wc0, degp):
    return pl.pallas_call(
        _tc1_body,
        grid=(GRID,),
        in_specs=[_row_spec(), _full((HID, HID)), _DEGP_SPEC],
        out_specs=_row_spec(),
        out_shape=jax.ShapeDtypeStruct((N_NODES, HID), jnp.float32),
    )(h0, wc0, degp)


def _tc_mid(parts, u, hprev, degp, scv, shv, wnext, residual):
    body = functools.partial(_tc_mid_body, residual=residual)
    return pl.pallas_call(
        body,
        grid=(GRID,),
        in_specs=[_PARTS_SPEC, _row_spec(), _row_spec(), _DEGP_SPEC,
                  _full((1, HID)), _full((1, HID)), _full((HID, HID))],
        out_specs=[_row_spec(), _row_spec()],
        out_shape=[jax.ShapeDtypeStruct((N_NODES, HID), jnp.float32),
                   jax.ShapeDtypeStruct((N_NODES, HID), jnp.float32)],
    )(parts, u, hprev, degp, scv, shv, wnext)


def _tc_fin(parts, u, hprev, degp, scv, shv, batch2, gfeat,
            wg1, bg1, wg2, bg2, wp1a, wp1b, bp1, wp2, bp2, wp3, bp3):
    h2 = HID // 2
    h4 = HID // 4
    return pl.pallas_call(
        _tc_fin_body,
        grid=(GRID,),
        in_specs=[_PARTS_SPEC, _row_spec(), _row_spec(), _DEGP_SPEC,
                  _full((1, HID)), _full((1, HID)),
                  pl.BlockSpec((BLK, 1), lambda i: (i, 0)),
                  _full((N_GRAPHS, GFD)),
                  _full((GFD, h2)), _full((1, h2)),
                  _full((h2, h4)), _full((1, h4)),
                  _full((HID, h2)), _full((h4, h2)), _full((1, h2)),
                  _full((h2, h4)), _full((1, h4)),
                  _full((h4, 1)), _full((1, 1))],
        out_specs=[_full((N_GRAPHS, HID)), _full((N_GRAPHS, HID)),
                   _full((N_GRAPHS, 1))],
        out_shape=[jax.ShapeDtypeStruct((N_GRAPHS, HID), jnp.float32),
                   jax.ShapeDtypeStruct((N_GRAPHS, HID), jnp.float32),
                   jax.ShapeDtypeStruct((N_GRAPHS, 1), jnp.float32)],
    )(parts, u, hprev, degp, scv, shv, batch2, gfeat,
      wg1, bg1, wg2, bg2, wp1a, wp1b, bp1, wp2, bp2, wp3, bp3)[2]


def kernel(x, edge_index, batch, graph_features, W_emb, b_emb, Wc, bc,
           bn_g, bn_b, Wg1, bg1, Wg2, bg2, Wp1, bp1, Wp2, bp2, Wp3, bp3):
    src1 = edge_index[0].astype(jnp.int32)
    dst1 = edge_index[1].astype(jnp.int32)
    batch2 = batch.astype(jnp.int32).reshape(N_NODES, 1)

    inv = 1.0 / jnp.sqrt(1.0 + BN_EPS)
    scv = (inv * bn_g).reshape(3, 1, HID)
    shv = (bc * inv * bn_g + bn_b).reshape(3, 1, HID)

    degp = _deg_sc(dst1, jnp.ones((CHUNK, HID), jnp.float32),
                   jnp.zeros((CHUNK, HID), jnp.float32)).reshape(2, NNP, HID)

    zrows = jnp.zeros((CHUNK, HID), jnp.float32)
    h0 = _tc_emb(x, W_emb, b_emb.reshape(1, HID))
    u1 = _tc1(h0, Wc[0], degp)
    p1 = _agg_sc(u1, src1, dst1, zrows).reshape(2, NNP, HID)
    h1, u2 = _tc_mid(p1, u1, u1, degp, scv[0], shv[0], Wc[1], residual=False)
    p2 = _agg_sc(u2, src1, dst1, zrows).reshape(2, NNP, HID)
    h2, u3 = _tc_mid(p2, u2, h1, degp, scv[1], shv[1], Wc[2], residual=True)
    p3 = _agg_sc(u3, src1, dst1, zrows).reshape(2, NNP, HID)

    return _tc_fin(p3, u3, h2, degp, scv[2], shv[2], batch2, graph_features,
                   Wg1, bg1.reshape(1, HID // 2), Wg2, bg2.reshape(1, HID // 4),
                   Wp1[:HID], Wp1[HID:], bp1.reshape(1, HID // 2),
                   Wp2, bp2.reshape(1, HID // 4), Wp3, bp3.reshape(1, 1))
